# baseline (speedup 1.0000x reference)
import functools

import jax
import jax.numpy as jnp
import numpy as np
from jax import lax
from jax.experimental import pallas as pl
from jax.experimental.pallas import tpu as pltpu
from jax.experimental.pallas import tpu_sc as plsc

SENSOR_COUNT = 4096
_B, _C, _D0, _D1 = 64, 2, 512, 512
_PLANES = _B * _C
_PLANE = _D0 * _D1
_NW = 32
_PPW = _PLANES // _NW
_SPW = SENSOR_COUNT * _PPW
_ZW = 65536
_ZPW = _PLANE * _PPW // _ZW


def _precompute():
    with jax.default_device(jax.local_devices(backend="cpu")[0]):
        perm = jax.random.permutation(jax.random.key(42), _D0 * _D1)
        dim_inds = np.asarray(perm[:SENSOR_COUNT]).astype(np.int32)

    n = SENSOR_COUNT * _B
    d0i = dim_inds // _D1
    d1i = dim_inds % _D1
    r = np.arange(2 * n, dtype=np.int32)
    indices = np.stack(
        [(r % n) // SENSOR_COUNT, r // n,
         np.tile(d0i, 2 * _B), np.tile(d1i, 2 * _B)], axis=1)

    plane_off = (np.arange(_PLANES, dtype=np.int32) * _PLANE)[:, None]
    gidx = (plane_off + dim_inds[None, :]).reshape(_NW, _SPW)
    return indices, gidx


_INDICES, _GIDX = _precompute()
_ZCONST = np.zeros((_ZW,), np.float32)

_mesh = plsc.VectorSubcoreMesh(core_axis_name="c", subcore_axis_name="s",
                               num_cores=2, num_subcores=16)


@functools.partial(
    pl.kernel,
    out_type=jax.ShapeDtypeStruct((_PLANES * _PLANE,), jnp.float32),
    mesh=_mesh,
    scratch_types=[
        pltpu.VMEM((_SPW,), jnp.int32),
        pltpu.VMEM((_SPW,), jnp.float32),
        pltpu.VMEM((_ZW,), jnp.float32),
        pltpu.SemaphoreType.DMA,
    ],
)
def _sc_kernel(x_hbm, gidx_hbm, zeros_hbm, out_hbm, idx_v, vals_v, zero_v,
               sem):
    wid = lax.axis_index("s") * 2 + lax.axis_index("c")
    pltpu.sync_copy(zeros_hbm, zero_v)
    pltpu.sync_copy(gidx_hbm.at[wid], idx_v)
    gather = pltpu.async_copy(x_hbm.at[idx_v], vals_v, sem)
    base = wid * _PPW * _PLANE
    for k in range(_ZPW):
        pltpu.sync_copy(zero_v, out_hbm.at[pl.ds(base + k * _ZW, _ZW)])
    gather.wait()
    pltpu.async_copy(vals_v, out_hbm.at[idx_v], sem).wait()


def kernel(data_batch):
    b, c, d0, d1 = data_batch.shape
    x = data_batch.reshape(-1)
    out = _sc_kernel(x, jnp.asarray(_GIDX), jnp.asarray(_ZCONST))
    values = out.reshape(b, c, d0, d1)
    return values, jnp.asarray(_INDICES)
